# baseline (speedup 1.0000x reference)
import jax
import jax.numpy as jnp
from jax import lax
from jax.experimental import pallas as pl
from jax.experimental.pallas import tpu as pltpu
from jax.experimental.pallas import tpu_sc as plsc

_NC = 2
_NS = 16
_L = 16
_NW = _NC * _NS

_N = 4194304
_CHUNK = _N // _NW
_S = 8192
_NSUB = _CHUNK // _S
_BINS = 2048
_SHIFT1 = 20
_SHIFT2 = 9
_U = 8

_mesh = plsc.VectorSubcoreMesh(core_axis_name="c", subcore_axis_name="s")

_OUT = jax.ShapeDtypeStruct((_NW * 2 * _BINS,), jnp.float32)

_SCRATCH = [
    pltpu.VMEM((_S,), jnp.float32),
    pltpu.VMEM((_S,), jnp.float32),
    pltpu.VMEM((_S,), jnp.float32),
    pltpu.VMEM((_S,), jnp.float32),
    pltpu.VMEM((_L * _BINS,), jnp.float32),
    pltpu.VMEM((_L * _BINS,), jnp.float32),
    pltpu.VMEM((2 * _BINS,), jnp.float32),
    pltpu.SemaphoreType.DMA,
    pltpu.SemaphoreType.DMA,
    pltpu.SemaphoreType.DMA,
    pltpu.SemaphoreType.DMA,
]


def _histogram_pass(o_hbm, t_hbm, bvec, out_hbm,
                    o0, o1, t0, t1, hc, hs, stage, so0, so1, st0, st1,
                    second):
    wid = lax.axis_index("s") * _NC + lax.axis_index("c")
    base = wid * _CHUNK

    zeros = jnp.zeros((_L,), jnp.float32)
    ones = jnp.ones((_L,), jnp.float32)
    lane = lax.iota(jnp.int32, _L)
    gath = lax.iota(jnp.int32, _L) * _L

    obufs, tbufs = (o0, o1), (t0, t1)
    osems, tsems = (so0, so1), (st0, st1)

    def start(j):
        off = base + j * _S
        co = pltpu.async_copy(o_hbm.at[pl.ds(off, _S)], obufs[j % 2], osems[j % 2])
        ct = pltpu.async_copy(t_hbm.at[pl.ds(off, _S)], tbufs[j % 2], tsems[j % 2])
        return co, ct

    pend = start(0)

    def zero_body(v, c):
        off = pl.multiple_of(v * (_L * _L), _L)
        for h in range(_L):
            hc[pl.ds(off + h * _L, _L)] = zeros
            hs[pl.ds(off + h * _L, _L)] = zeros
        return c

    lax.fori_loop(0, _BINS // _L, zero_body, 0)

    def compute(ob, tb):
        @plsc.parallel_loop(0, _S, _L, unroll=_U)
        def body(i):
            off = pl.multiple_of(i, _L)
            o = ob[pl.ds(off, _L)]
            t = tb[pl.ds(off, _L)]
            r = (t - o) / t
            e = r * r
            u = lax.bitcast_convert_type(e, jnp.int32)
            if second:
                coarse = jnp.bitwise_and(
                    lax.shift_right_logical(u, _SHIFT1), _BINS - 1)
                sub = jnp.bitwise_and(
                    lax.shift_right_logical(u, _SHIFT2), _BINS - 1)
                m = coarse == bvec
                idx = lax.shift_left(sub, 4) + lane
                plsc.addupdate_scatter(hc, [idx], ones, mask=m)
                plsc.addupdate_scatter(hs, [idx], e, mask=m)
            else:
                coarse = jnp.bitwise_and(
                    lax.shift_right_logical(u, _SHIFT1), _BINS - 1)
                idx = lax.shift_left(coarse, 4) + lane
                plsc.addupdate_scatter(hc, [idx], ones)
                plsc.addupdate_scatter(hs, [idx], e)

    for j in range(_NSUB):
        nxt = start(j + 1) if j + 1 < _NSUB else None
        co, ct = pend
        co.wait()
        ct.wait()
        compute(obufs[j % 2], tbufs[j % 2])
        pend = nxt

    def red_body(v, c):
        off = pl.multiple_of(v * _L, _L)
        base = v * (_L * _L)
        acc_c = plsc.load_gather(hc, [gath + base])
        acc_s = plsc.load_gather(hs, [gath + base])
        for h in range(1, _L):
            acc_c = acc_c + plsc.load_gather(hc, [gath + (base + h)])
            acc_s = acc_s + plsc.load_gather(hs, [gath + (base + h)])
        stage[pl.ds(off, _L)] = acc_c
        stage[pl.ds(_BINS + off, _L)] = acc_s
        return c

    lax.fori_loop(0, _BINS // _L, red_body, 0)

    pltpu.sync_copy(stage, out_hbm.at[pl.ds(wid * 2 * _BINS, 2 * _BINS)])


def _pass1_body(o_hbm, t_hbm, out_hbm,
                o0, o1, t0, t1, hc, hs, stage, so0, so1, st0, st1):
    _histogram_pass(o_hbm, t_hbm, None, out_hbm,
                    o0, o1, t0, t1, hc, hs, stage, so0, so1, st0, st1,
                    second=False)


def _pass2_body(o_hbm, t_hbm, b_hbm, out_hbm,
                o0, o1, t0, t1, hc, hs, stage, so0, so1, st0, st1,
                bbuf, sb):
    pltpu.async_copy(b_hbm, bbuf, sb).wait()
    bvec = bbuf[...]
    _histogram_pass(o_hbm, t_hbm, bvec, out_hbm,
                    o0, o1, t0, t1, hc, hs, stage, so0, so1, st0, st1,
                    second=True)


_params = pltpu.CompilerParams(needs_layout_passes=False)

_pass1 = pl.kernel(_pass1_body, out_type=_OUT, mesh=_mesh,
                   scratch_types=list(_SCRATCH), compiler_params=_params)
_pass2 = pl.kernel(_pass2_body, out_type=_OUT, mesh=_mesh,
                   scratch_types=list(_SCRATCH)
                   + [pltpu.VMEM((_L,), jnp.int32), pltpu.SemaphoreType.DMA],
                   compiler_params=_params)


def _locate(counts, sums, want):
    ccum = jnp.cumsum(counts)
    scum = jnp.cumsum(sums)
    idx = jnp.argmax(ccum >= want)
    return idx, ccum[idx] - counts[idx], scum[idx] - sums[idx]


def kernel(output, target):
    n = output.shape[0]
    k = int(n * 0.97)

    h1 = _pass1(output, target).reshape(_NW, 2, _BINS)
    tot1 = h1.sum(axis=0)
    kf = jnp.float32(k)
    b, below_c, below_s = _locate(tot1[0], tot1[1], kf)

    bvec = jnp.full((_L,), b.astype(jnp.int32), dtype=jnp.int32)
    h2 = _pass2(output, target, bvec).reshape(_NW, 2, _BINS)
    tot2 = h2.sum(axis=0)
    rem = kf - below_c
    t2, below_c2, below_s2 = _locate(tot2[0], tot2[1], rem)

    taken = rem - below_c2
    mean_t2 = tot2[1][t2] / jnp.maximum(tot2[0][t2], 1.0)
    total = below_s + below_s2 + taken * mean_t2
    return total / kf


# speedup vs baseline: 1.1895x; 1.1895x over previous
import jax
import jax.numpy as jnp
from jax import lax
from jax.experimental import pallas as pl
from jax.experimental.pallas import tpu as pltpu
from jax.experimental.pallas import tpu_sc as plsc

_NC = 2
_NS = 16
_L = 16
_NW = _NC * _NS

_N = 4194304
_CHUNK = _N // _NW
_S = 16384
_NSUB = _CHUNK // _S
_BINS = 1024
_SHIFT1 = 21
_SHIFT2 = 11
_U = 4

_mesh = plsc.VectorSubcoreMesh(core_axis_name="c", subcore_axis_name="s")

_OUT = jax.ShapeDtypeStruct((_NW * 2 * _BINS,), jnp.float32)

_SCRATCH = [
    pltpu.VMEM((_S,), jnp.float32),
    pltpu.VMEM((_S,), jnp.float32),
    pltpu.VMEM((_S,), jnp.float32),
    pltpu.VMEM((_S,), jnp.float32),
    pltpu.VMEM((_L * _BINS,), jnp.float32),
    pltpu.VMEM((_L * _BINS,), jnp.float32),
    pltpu.VMEM((2 * _BINS,), jnp.float32),
    pltpu.SemaphoreType.DMA,
    pltpu.SemaphoreType.DMA,
    pltpu.SemaphoreType.DMA,
    pltpu.SemaphoreType.DMA,
]


def _histogram_pass(o_hbm, t_hbm, bvec, out_hbm,
                    o0, o1, t0, t1, hc, hs, stage, so0, so1, st0, st1,
                    second):
    wid = lax.axis_index("s") * _NC + lax.axis_index("c")
    base = wid * _CHUNK

    zeros = jnp.zeros((_L,), jnp.float32)
    ones = jnp.ones((_L,), jnp.float32)
    lane = lax.iota(jnp.int32, _L)
    gath = lax.iota(jnp.int32, _L) * _L

    obufs, tbufs = (o0, o1), (t0, t1)
    osems, tsems = (so0, so1), (st0, st1)

    def start(j):
        off = base + j * _S
        co = pltpu.async_copy(o_hbm.at[pl.ds(off, _S)], obufs[j % 2], osems[j % 2])
        ct = pltpu.async_copy(t_hbm.at[pl.ds(off, _S)], tbufs[j % 2], tsems[j % 2])
        return co, ct

    pend = start(0)

    def zero_body(v, c):
        off = pl.multiple_of(v * (_L * _L), _L)
        for h in range(_L):
            hc[pl.ds(off + h * _L, _L)] = zeros
            hs[pl.ds(off + h * _L, _L)] = zeros
        return c

    lax.fori_loop(0, _BINS // _L, zero_body, 0)

    def compute(ob, tb):
        @plsc.parallel_loop(0, _S, _L, unroll=_U)
        def body(i):
            off = pl.multiple_of(i, _L)
            o = ob[pl.ds(off, _L)]
            t = tb[pl.ds(off, _L)]
            r = (t - o) / t
            e = r * r
            u = lax.bitcast_convert_type(e, jnp.int32)
            if second:
                coarse = jnp.bitwise_and(
                    lax.shift_right_logical(u, _SHIFT1), _BINS - 1)
                sub = jnp.bitwise_and(
                    lax.shift_right_logical(u, _SHIFT2), _BINS - 1)
                m = coarse == bvec
                idx = lax.shift_left(sub, 4) + lane
                plsc.addupdate_scatter(hc, [idx], ones, mask=m)
                plsc.addupdate_scatter(hs, [idx], e, mask=m)
            else:
                coarse = jnp.bitwise_and(
                    lax.shift_right_logical(u, _SHIFT1), _BINS - 1)
                idx = lax.shift_left(coarse, 4) + lane
                plsc.addupdate_scatter(hc, [idx], ones)
                plsc.addupdate_scatter(hs, [idx], e)

    for j in range(_NSUB):
        nxt = start(j + 1) if j + 1 < _NSUB else None
        co, ct = pend
        co.wait()
        ct.wait()
        compute(obufs[j % 2], tbufs[j % 2])
        pend = nxt

    def red_body(v, c):
        off = pl.multiple_of(v * _L, _L)
        base = v * (_L * _L)
        acc_c = plsc.load_gather(hc, [gath + base])
        acc_s = plsc.load_gather(hs, [gath + base])
        for h in range(1, _L):
            acc_c = acc_c + plsc.load_gather(hc, [gath + (base + h)])
            acc_s = acc_s + plsc.load_gather(hs, [gath + (base + h)])
        stage[pl.ds(off, _L)] = acc_c
        stage[pl.ds(_BINS + off, _L)] = acc_s
        return c

    lax.fori_loop(0, _BINS // _L, red_body, 0)

    pltpu.sync_copy(stage, out_hbm.at[pl.ds(wid * 2 * _BINS, 2 * _BINS)])


def _pass1_body(o_hbm, t_hbm, out_hbm,
                o0, o1, t0, t1, hc, hs, stage, so0, so1, st0, st1):
    _histogram_pass(o_hbm, t_hbm, None, out_hbm,
                    o0, o1, t0, t1, hc, hs, stage, so0, so1, st0, st1,
                    second=False)


def _pass2_body(o_hbm, t_hbm, b_hbm, out_hbm,
                o0, o1, t0, t1, hc, hs, stage, so0, so1, st0, st1,
                bbuf, sb):
    pltpu.async_copy(b_hbm, bbuf, sb).wait()
    bvec = bbuf[...]
    _histogram_pass(o_hbm, t_hbm, bvec, out_hbm,
                    o0, o1, t0, t1, hc, hs, stage, so0, so1, st0, st1,
                    second=True)


_params = pltpu.CompilerParams(needs_layout_passes=False)

_pass1 = pl.kernel(_pass1_body, out_type=_OUT, mesh=_mesh,
                   scratch_types=list(_SCRATCH), compiler_params=_params)
_pass2 = pl.kernel(_pass2_body, out_type=_OUT, mesh=_mesh,
                   scratch_types=list(_SCRATCH)
                   + [pltpu.VMEM((_L,), jnp.int32), pltpu.SemaphoreType.DMA],
                   compiler_params=_params)


def _locate(counts, sums, want):
    ccum = jnp.cumsum(counts)
    scum = jnp.cumsum(sums)
    idx = jnp.argmax(ccum >= want)
    return idx, ccum[idx] - counts[idx], scum[idx] - sums[idx]


def kernel(output, target):
    n = output.shape[0]
    k = int(n * 0.97)

    h1 = _pass1(output, target).reshape(_NW, 2, _BINS)
    tot1 = h1.sum(axis=0)
    kf = jnp.float32(k)
    b, below_c, below_s = _locate(tot1[0], tot1[1], kf)

    bvec = jnp.full((_L,), b.astype(jnp.int32), dtype=jnp.int32)
    h2 = _pass2(output, target, bvec).reshape(_NW, 2, _BINS)
    tot2 = h2.sum(axis=0)
    rem = kf - below_c
    t2, below_c2, below_s2 = _locate(tot2[0], tot2[1], rem)

    taken = rem - below_c2
    mean_t2 = tot2[1][t2] / jnp.maximum(tot2[0][t2], 1.0)
    total = below_s + below_s2 + taken * mean_t2
    return total / kf
